# Initial kernel scaffold; baseline (speedup 1.0000x reference)
import jax
import jax.numpy as jnp
from jax.experimental import pallas as pl


def kernel(x, edge_index, edge_attr, Wp, bp, W0a, b0a, W0b, b0b, g0, be0, W1a, b1a, W1b, b1b, g1, be1, Wm1, bm1, Wm2, bm2):
    raise NotImplementedError("write your pallas kernel here")


# speedup vs baseline: 1.0023x; 1.0023x over previous
import functools
import jax
import jax.numpy as jnp
from jax import lax
from jax.experimental import pallas as pl
from jax.experimental.pallas import tpu as pltpu

_N = 10000
_E = 320000
_HID = 64
_EPS = 1e-5
_NEG = -3.0e38


def _node0_body(x_ref, wp_ref, bp_ref, wad_ref, was_ref, pd_ref, ps_ref):
    h = jnp.maximum(jnp.dot(x_ref[...], wp_ref[...],
                            preferred_element_type=jnp.float32) + bp_ref[...], 0.0)
    pd_ref[...] = jnp.dot(h, wad_ref[...], preferred_element_type=jnp.float32)
    ps_ref[...] = jnp.dot(h, was_ref[...], preferred_element_type=jnp.float32)


def _node0(x, Wp, bp, Wad, Was):
    out = [jax.ShapeDtypeStruct((_N, _HID), jnp.float32)] * 2
    return pl.pallas_call(
        _node0_body,
        out_shape=out,
    )(x, Wp, bp.reshape(1, -1), Wad, Was)


def _bn_next_body(agg_ref, g_ref, be_ref, wad_ref, was_ref, pd_ref, ps_ref):
    a = agg_ref[...]
    a = jnp.where(a > _NEG * 0.5, a, 0.0)
    mu = jnp.mean(a, axis=0, keepdims=True)
    var = jnp.mean((a - mu) * (a - mu), axis=0, keepdims=True)
    h = g_ref[...] * (a - mu) * jax.lax.rsqrt(var + _EPS) + be_ref[...]
    h = jnp.maximum(h, 0.0)
    pd_ref[...] = jnp.dot(h, wad_ref[...], preferred_element_type=jnp.float32)
    ps_ref[...] = jnp.dot(h, was_ref[...], preferred_element_type=jnp.float32)


def _bn_next(agg, g, be, Wad, Was):
    out = [jax.ShapeDtypeStruct((_N, _HID), jnp.float32)] * 2
    return pl.pallas_call(
        _bn_next_body,
        out_shape=out,
    )(agg, g.reshape(1, -1), be.reshape(1, -1), Wad, Was)


def _bn_final_body(agg_ref, g_ref, be_ref, wm1_ref, bm1_ref, wm2_ref, bm2_ref,
                   out_ref):
    a = agg_ref[...]
    a = jnp.where(a > _NEG * 0.5, a, 0.0)
    mu = jnp.mean(a, axis=0, keepdims=True)
    var = jnp.mean((a - mu) * (a - mu), axis=0, keepdims=True)
    h = g_ref[...] * (a - mu) * jax.lax.rsqrt(var + _EPS) + be_ref[...]
    h = jnp.maximum(h, 0.0)
    t = jnp.maximum(jnp.dot(h, wm1_ref[...],
                            preferred_element_type=jnp.float32) + bm1_ref[...], 0.0)
    out_ref[...] = jnp.dot(t, wm2_ref[...],
                           preferred_element_type=jnp.float32) + bm2_ref[...]


def _bn_final(agg, g, be, Wm1, bm1, Wm2, bm2):
    return pl.pallas_call(
        _bn_final_body,
        out_shape=jax.ShapeDtypeStruct((_N, Wm2.shape[1]), jnp.float32),
    )(agg, g.reshape(1, -1), be.reshape(1, -1), Wm1, bm1.reshape(1, -1),
      Wm2, bm2.reshape(1, -1))


def _edge_mlp_body(t_ref, ea_ref, wae_ref, ba_ref, wb_ref, bb_ref, m_ref):
    pre = t_ref[...] + jnp.dot(ea_ref[...], wae_ref[...],
                               preferred_element_type=jnp.float32) + ba_ref[...]
    pre = jnp.maximum(pre, 0.0)
    m_ref[...] = jnp.dot(pre, wb_ref[...],
                         preferred_element_type=jnp.float32) + bb_ref[...]


def _edge_mlp(t, ea, Wae, ba, Wb, bb, block=8000):
    grid = _E // block
    return pl.pallas_call(
        _edge_mlp_body,
        grid=(grid,),
        in_specs=[
            pl.BlockSpec((block, _HID), lambda i: (i, 0)),
            pl.BlockSpec((block, ea.shape[1]), lambda i: (i, 0)),
            pl.BlockSpec(Wae.shape, lambda i: (0, 0)),
            pl.BlockSpec((1, _HID), lambda i: (0, 0)),
            pl.BlockSpec(Wb.shape, lambda i: (0, 0)),
            pl.BlockSpec((1, _HID), lambda i: (0, 0)),
        ],
        out_specs=pl.BlockSpec((block, _HID), lambda i: (i, 0)),
        out_shape=jax.ShapeDtypeStruct((_E, _HID), jnp.float32),
    )(t, ea, Wae, ba.reshape(1, -1), Wb, bb.reshape(1, -1))


def _gather_add(pd, ps, dst, src):
    return pd[dst] + ps[src]


def _segment_max(m, dst):
    return jax.ops.segment_max(m, dst, num_segments=_N)


def kernel(x, edge_index, edge_attr, Wp, bp, W0a, b0a, W0b, b0b, g0, be0,
           W1a, b1a, W1b, b1b, g1, be1, Wm1, bm1, Wm2, bm2):
    src = edge_index[0]
    dst = edge_index[1]
    H = _HID

    pd0, ps0 = _node0(x, Wp, bp, W0a[:H], W0a[H:2 * H])

    t0 = _gather_add(pd0, ps0, dst, src)
    m0 = _edge_mlp(t0, edge_attr, W0a[2 * H:], b0a, W0b, b0b)
    agg0 = _segment_max(m0, dst)

    pd1, ps1 = _bn_next(agg0, g0, be0, W1a[:H], W1a[H:2 * H])
    t1 = _gather_add(pd1, ps1, dst, src)
    m1 = _edge_mlp(t1, edge_attr, W1a[2 * H:], b1a, W1b, b1b)
    agg1 = _segment_max(m1, dst)

    return _bn_final(agg1, g1, be1, Wm1, bm1, Wm2, bm2)


# speedup vs baseline: 1.6326x; 1.6289x over previous
import functools
import jax
import jax.numpy as jnp
from jax import lax
from jax.experimental import pallas as pl
from jax.experimental.pallas import tpu as pltpu
from jax.experimental.pallas import tpu_sc as plsc

_N = 10000
_E = 320000
_HID = 64
_EPS = 1e-5
_NEG = -3.0e38


def _node0_body(x_ref, wp_ref, bp_ref, wad_ref, was_ref, pd_ref, ps_ref):
    h = jnp.maximum(jnp.dot(x_ref[...], wp_ref[...],
                            preferred_element_type=jnp.float32) + bp_ref[...], 0.0)
    pd_ref[...] = jnp.dot(h, wad_ref[...], preferred_element_type=jnp.float32)
    ps_ref[...] = jnp.dot(h, was_ref[...], preferred_element_type=jnp.float32)


def _node0(x, Wp, bp, Wad, Was):
    out = [jax.ShapeDtypeStruct((_N, _HID), jnp.float32)] * 2
    return pl.pallas_call(
        _node0_body,
        out_shape=out,
    )(x, Wp, bp.reshape(1, -1), Wad, Was)


def _bn_next_body(agg_ref, g_ref, be_ref, wad_ref, was_ref, pd_ref, ps_ref):
    a = agg_ref[...]
    a = jnp.where(a > _NEG * 0.5, a, 0.0)
    mu = jnp.mean(a, axis=0, keepdims=True)
    var = jnp.mean((a - mu) * (a - mu), axis=0, keepdims=True)
    h = g_ref[...] * (a - mu) * jax.lax.rsqrt(var + _EPS) + be_ref[...]
    h = jnp.maximum(h, 0.0)
    pd_ref[...] = jnp.dot(h, wad_ref[...], preferred_element_type=jnp.float32)
    ps_ref[...] = jnp.dot(h, was_ref[...], preferred_element_type=jnp.float32)


def _bn_next(agg, g, be, Wad, Was):
    out = [jax.ShapeDtypeStruct((_N, _HID), jnp.float32)] * 2
    return pl.pallas_call(
        _bn_next_body,
        out_shape=out,
    )(agg, g.reshape(1, -1), be.reshape(1, -1), Wad, Was)


def _bn_final_body(agg_ref, g_ref, be_ref, wm1_ref, bm1_ref, wm2_ref, bm2_ref,
                   out_ref):
    a = agg_ref[...]
    a = jnp.where(a > _NEG * 0.5, a, 0.0)
    mu = jnp.mean(a, axis=0, keepdims=True)
    var = jnp.mean((a - mu) * (a - mu), axis=0, keepdims=True)
    h = g_ref[...] * (a - mu) * jax.lax.rsqrt(var + _EPS) + be_ref[...]
    h = jnp.maximum(h, 0.0)
    t = jnp.maximum(jnp.dot(h, wm1_ref[...],
                            preferred_element_type=jnp.float32) + bm1_ref[...], 0.0)
    out_ref[...] = jnp.dot(t, wm2_ref[...],
                           preferred_element_type=jnp.float32) + bm2_ref[...]


def _bn_final(agg, g, be, Wm1, bm1, Wm2, bm2):
    return pl.pallas_call(
        _bn_final_body,
        out_shape=jax.ShapeDtypeStruct((_N, Wm2.shape[1]), jnp.float32),
    )(agg, g.reshape(1, -1), be.reshape(1, -1), Wm1, bm1.reshape(1, -1),
      Wm2, bm2.reshape(1, -1))


def _edge_mlp_body(gd_ref, gs_ref, ea_ref, wae_ref, ba_ref, wb_ref, bb_ref,
                   m_ref):
    pre = gd_ref[...] + gs_ref[...] + jnp.dot(
        ea_ref[...], wae_ref[...],
        preferred_element_type=jnp.float32) + ba_ref[...]
    pre = jnp.maximum(pre, 0.0)
    m_ref[...] = jnp.dot(pre, wb_ref[...],
                         preferred_element_type=jnp.float32) + bb_ref[...]


def _edge_mlp(gd, gs, ea, Wae, ba, Wb, bb, block=8000):
    grid = _E // block
    return pl.pallas_call(
        _edge_mlp_body,
        grid=(grid,),
        in_specs=[
            pl.BlockSpec((block, _HID), lambda i: (i, 0)),
            pl.BlockSpec((block, _HID), lambda i: (i, 0)),
            pl.BlockSpec((block, ea.shape[1]), lambda i: (i, 0)),
            pl.BlockSpec(Wae.shape, lambda i: (0, 0)),
            pl.BlockSpec((1, _HID), lambda i: (0, 0)),
            pl.BlockSpec(Wb.shape, lambda i: (0, 0)),
            pl.BlockSpec((1, _HID), lambda i: (0, 0)),
        ],
        out_specs=pl.BlockSpec((block, _HID), lambda i: (i, 0)),
        out_shape=jax.ShapeDtypeStruct((_E, _HID), jnp.float32),
    )(gd, gs, ea, Wae, ba.reshape(1, -1), Wb, bb.reshape(1, -1))


_NC = 2
_NS = 16
_NW = _NC * _NS
_BPW = _E // _NW
_CH = 400
_NCHUNK = _BPW // _CH


def _sc_gather2(pd, ps, dst, src):
    mesh = plsc.VectorSubcoreMesh(core_axis_name="c", subcore_axis_name="s")

    @functools.partial(
        pl.kernel, mesh=mesh,
        compiler_params=pltpu.CompilerParams(use_tc_tiling_on_sc=False),
        out_type=[jax.ShapeDtypeStruct((_E, _HID), jnp.float32)] * 2,
        scratch_types=[
            pltpu.VMEM((_CH,), jnp.int32),
            pltpu.VMEM((_CH,), jnp.int32),
            pltpu.VMEM((_CH, _HID), jnp.float32),
            pltpu.VMEM((_CH, _HID), jnp.float32),
            pltpu.SemaphoreType.DMA,
            pltpu.SemaphoreType.DMA,
        ],
    )
    def k(pd_hbm, ps_hbm, dst_hbm, src_hbm, gd_hbm, gs_hbm,
          di_v, si_v, gd_v, gs_v, sem1, sem2):
        wid = lax.axis_index("s") * _NC + lax.axis_index("c")
        base = wid * _BPW

        @pl.loop(0, _NCHUNK)
        def _(j):
            off = base + j * _CH
            pltpu.sync_copy(dst_hbm.at[pl.ds(off, _CH)], di_v)
            pltpu.sync_copy(src_hbm.at[pl.ds(off, _CH)], si_v)
            a = pltpu.async_copy(pd_hbm.at[di_v], gd_v, sem1)
            b = pltpu.async_copy(ps_hbm.at[si_v], gs_v, sem2)
            a.wait()
            b.wait()
            pltpu.sync_copy(gd_v, gd_hbm.at[pl.ds(off, _CH)])
            pltpu.sync_copy(gs_v, gs_hbm.at[pl.ds(off, _CH)])

    return k(pd, ps, dst, src)


def _segment_max(m, dst):
    return jax.ops.segment_max(m, dst, num_segments=_N)


def kernel(x, edge_index, edge_attr, Wp, bp, W0a, b0a, W0b, b0b, g0, be0,
           W1a, b1a, W1b, b1b, g1, be1, Wm1, bm1, Wm2, bm2):
    src = edge_index[0]
    dst = edge_index[1]
    H = _HID

    pd0, ps0 = _node0(x, Wp, bp, W0a[:H], W0a[H:2 * H])

    gd0, gs0 = _sc_gather2(pd0, ps0, dst, src)
    m0 = _edge_mlp(gd0, gs0, edge_attr, W0a[2 * H:], b0a, W0b, b0b)
    agg0 = _segment_max(m0, dst)

    pd1, ps1 = _bn_next(agg0, g0, be0, W1a[:H], W1a[H:2 * H])
    gd1, gs1 = _sc_gather2(pd1, ps1, dst, src)
    m1 = _edge_mlp(gd1, gs1, edge_attr, W1a[2 * H:], b1a, W1b, b1b)
    agg1 = _segment_max(m1, dst)

    return _bn_final(agg1, g1, be1, Wm1, bm1, Wm2, bm2)
